# baseline (speedup 1.0000x reference)
import functools

import jax
import jax.numpy as jnp
from jax import lax
from jax.experimental import pallas as pl
from jax.experimental.pallas import tpu as pltpu
from jax.experimental.pallas import tpu_sc as plsc

VOCAB = 1000000
EMBED = 64
HIDDEN = 512
NUM_CLASSES = 128
BATCH = 4096
SEQ = 200

_CHUNK = 100
_CHUNKS_PER_ROW = SEQ // _CHUNK
_EG = EMBED // 16


def _make_sc_pool():
    info = plsc.get_sparse_core_info()
    nc, ns = info.num_cores, info.num_subcores
    nw = nc * ns
    rows_per_w = BATCH // nw
    chunks_per_w = rows_per_w * _CHUNKS_PER_ROW

    mesh = plsc.VectorSubcoreMesh(core_axis_name="c", subcore_axis_name="s")

    @functools.partial(
        pl.kernel,
        mesh=mesh,
        compiler_params=pltpu.CompilerParams(use_tc_tiling_on_sc=False),
        out_type=jax.ShapeDtypeStruct((BATCH, EMBED), jnp.float32),
        scratch_types=[
            pltpu.VMEM((chunks_per_w, _CHUNK), jnp.int32),
            pltpu.VMEM((_CHUNK, EMBED), jnp.float32),
            pltpu.VMEM((_CHUNK, EMBED), jnp.float32),
            pltpu.VMEM((rows_per_w, EMBED), jnp.float32),
            pltpu.SemaphoreType.DMA,
            pltpu.SemaphoreType.DMA,
        ],
    )
    def sc_pool(x_hbm, table_hbm, out_hbm, idx_v, rows0, rows1, out_v,
                sem0, sem1):
        wid = lax.axis_index("s") * nc + lax.axis_index("c")
        chunk0 = wid * chunks_per_w
        row0 = wid * rows_per_w

        pltpu.sync_copy(x_hbm.at[pl.ds(chunk0, chunks_per_w)], idx_v)

        def fire(c, buf, sem):
            pltpu.async_copy(table_hbm.at[idx_v.at[c]], buf, sem)

        def wait(buf, sem):
            pltpu.make_async_copy(table_hbm.at[idx_v.at[0]], buf, sem).wait()

        def sum_chunk(buf, acc):
            def body(r, a):
                return tuple(
                    a[g] + buf[r, pl.ds(16 * g, 16)] for g in range(_EG)
                )
            return lax.fori_loop(0, _CHUNK, body, acc, unroll=4)

        fire(0, rows0, sem0)

        def row_body(i, _):
            fire(2 * i + 1, rows1, sem1)
            wait(rows0, sem0)
            zero = jnp.zeros((16,), jnp.float32)
            acc = sum_chunk(rows0, (zero,) * _EG)

            @pl.when(i < rows_per_w - 1)
            def _():
                fire(2 * i + 2, rows0, sem0)

            wait(rows1, sem1)
            acc = sum_chunk(rows1, acc)
            for g in range(_EG):
                out_v[i, pl.ds(16 * g, 16)] = acc[g]
            return 0

        lax.fori_loop(0, rows_per_w, row_body, 0)

        pltpu.sync_copy(out_v, out_hbm.at[pl.ds(row0, rows_per_w)])

    return sc_pool


_sc_pool = None


def _mlp_body(p_ref, w1_ref, b1_ref, w2_ref, b2_ref, o_ref):
    p = p_ref[...] * (1.0 / SEQ)
    h = jnp.dot(p, w1_ref[...], preferred_element_type=jnp.float32)
    h = jnp.maximum(h + b1_ref[...], 0.0)
    o = jnp.dot(h, w2_ref[...], preferred_element_type=jnp.float32)
    o_ref[...] = o + b2_ref[...]


def _mlp(pooled, W1, b1, W2, b2):
    blk = 512
    grid = BATCH // blk
    return pl.pallas_call(
        _mlp_body,
        grid=(grid,),
        in_specs=[
            pl.BlockSpec((blk, EMBED), lambda i: (i, 0)),
            pl.BlockSpec((EMBED, HIDDEN), lambda i: (0, 0)),
            pl.BlockSpec((1, HIDDEN), lambda i: (0, 0)),
            pl.BlockSpec((HIDDEN, NUM_CLASSES), lambda i: (0, 0)),
            pl.BlockSpec((1, NUM_CLASSES), lambda i: (0, 0)),
        ],
        out_specs=pl.BlockSpec((blk, NUM_CLASSES), lambda i: (i, 0)),
        out_shape=jax.ShapeDtypeStruct((BATCH, NUM_CLASSES), jnp.float32),
    )(pooled, W1, b1.reshape(1, HIDDEN), W2, b2.reshape(1, NUM_CLASSES))


def kernel(x, table, W1, b1, W2, b2):
    global _sc_pool
    if _sc_pool is None:
        _sc_pool = _make_sc_pool()
    x_chunks = x.astype(jnp.int32).reshape(BATCH * _CHUNKS_PER_ROW, _CHUNK)
    pooled = _sc_pool(x_chunks, table)
    return _mlp(pooled, W1, b1, W2, b2)
